# Initial kernel scaffold; baseline (speedup 1.0000x reference)
import jax
import jax.numpy as jnp
from jax.experimental import pallas as pl


def kernel(inp, W_res, W_q, W_k, W_left, W_right, emb_in, emb_out):
    raise NotImplementedError("write your pallas kernel here")


# speedup vs baseline: 9.2558x; 9.2558x over previous
import jax
import jax.numpy as jnp
from jax.experimental import pallas as pl
from jax.experimental.pallas import tpu as pltpu

NHEAD = 8
QDIM = 512
TOPK = 8
NQ = 256
SEQ = 2048
INF = 1024
TB = 256
NTB = SEQ // TB
EMB_ROWS = (NQ - 1) * TOPK + (NQ - 1) + 1
EMB_PAD = 2304

_SQRT_2_OVER_PI = 0.7978845608028654


def _top8_desc(s, n):
    rows = s.shape[0]
    iota = jax.lax.broadcasted_iota(jnp.int32, (rows, n), 1)
    vals, idxs = [], []
    for _ in range(TOPK):
        m = jnp.max(s, axis=-1, keepdims=True)
        hit = s == m
        idx = jnp.min(jnp.where(hit, iota, n), axis=-1, keepdims=True)
        vals.append(m)
        idxs.append(idx)
        s = jnp.where(iota == idx, -jnp.inf, s)
    return jnp.concatenate(vals, axis=1), jnp.concatenate(idxs, axis=1)


def _gather8(table, sel):
    out = jnp.zeros(sel.shape, table.dtype)
    for a in range(TOPK):
        out = out + jnp.where(sel == a, table[:, a:a + 1], 0)
    return out


def _nt(a, b):
    return jax.lax.dot_general(a, b, (((1,), (1,)), ((), ())),
                               preferred_element_type=jnp.float32)


def _fused_body(inp_f32, inp_bf, wres, wq, wl, wr, wk, embin, embout,
                out_ref):
    h = pl.program_id(1)

    x = _nt(inp_bf[...], wq[0])
    xb = x.astype(jnp.bfloat16)
    sl = _nt(xb, wl[...])
    sr = _nt(xb, wr[...])

    lv, li = _top8_desc(sl, NQ)
    rv, ri = _top8_desc(sr, NQ)

    lrep = jnp.concatenate(
        [jnp.broadcast_to(lv[:, a:a + 1], (TB, TOPK)) for a in range(TOPK)],
        axis=1)
    rtil = jnp.concatenate([rv] * TOPK, axis=1)
    dot, cidx = _top8_desc(lrep + rtil, TOPK * TOPK)

    e = jnp.exp(dot - jnp.max(dot, axis=-1, keepdims=True))
    scores = e / jnp.sum(e, axis=-1, keepdims=True)

    a_sel = cidx // TOPK
    b_sel = jnp.remainder(cidx, TOPK)
    fi = _gather8(li, a_sel) * TOPK + _gather8(ri, b_sel)

    proj = _nt(inp_bf[...], wk[0])
    ad = _nt(proj.astype(jnp.bfloat16), embin[...])

    iota_e = jax.lax.broadcasted_iota(jnp.int32, (TB, EMB_PAD), 1)
    indots = []
    for k in range(TOPK):
        onek = iota_e == fi[:, k:k + 1]
        indots.append(jnp.sum(jnp.where(onek, ad, 0.0), axis=-1,
                              keepdims=True))
    in_dot = jnp.concatenate(indots, axis=1)

    g = 0.5 * in_dot * (1.0 + jnp.tanh(
        _SQRT_2_OVER_PI * (in_dot + 0.044715 * in_dot * in_dot * in_dot)))
    w = scores * g

    p = jnp.zeros((TB, EMB_PAD), jnp.float32)
    for k in range(TOPK):
        p = p + jnp.where(iota_e == fi[:, k:k + 1], w[:, k:k + 1], 0.0)

    outc = jax.lax.dot_general(p.astype(jnp.bfloat16), embout[...],
                               (((1,), (0,)), ((), ())),
                               preferred_element_type=jnp.float32)

    @pl.when(h == 0)
    def _():
        out_ref[...] = _nt(inp_f32[...], wres[...]) + outc

    @pl.when(h != 0)
    def _():
        out_ref[...] += outc


def kernel(inp, W_res, W_q, W_k, W_left, W_right, emb_in, emb_out):
    inp2d = inp.reshape(SEQ, INF)
    inp_bf = inp2d.astype(jnp.bfloat16)
    wq = W_q.reshape(NHEAD, QDIM, INF).astype(jnp.bfloat16)
    wk = W_k.reshape(NHEAD, INF, INF).astype(jnp.bfloat16)
    wl = W_left.astype(jnp.bfloat16)
    wr = W_right.astype(jnp.bfloat16)
    pad = EMB_PAD - EMB_ROWS
    embin = jnp.pad(emb_in[:EMB_ROWS].astype(jnp.bfloat16), ((0, pad), (0, 0)))
    embout = jnp.pad(emb_out[:EMB_ROWS].astype(jnp.bfloat16),
                     ((0, pad), (0, 0)))

    grid = (NTB, NHEAD)
    out = pl.pallas_call(
        _fused_body,
        grid=grid,
        in_specs=[
            pl.BlockSpec((TB, INF), lambda tb, h: (tb, 0)),
            pl.BlockSpec((TB, INF), lambda tb, h: (tb, 0)),
            pl.BlockSpec((INF, INF), lambda tb, h: (0, 0)),
            pl.BlockSpec((1, QDIM, INF), lambda tb, h: (h, 0, 0)),
            pl.BlockSpec((NQ, QDIM), lambda tb, h: (0, 0)),
            pl.BlockSpec((NQ, QDIM), lambda tb, h: (0, 0)),
            pl.BlockSpec((1, INF, INF), lambda tb, h: (h, 0, 0)),
            pl.BlockSpec((EMB_PAD, INF), lambda tb, h: (0, 0)),
            pl.BlockSpec((EMB_PAD, INF), lambda tb, h: (0, 0)),
        ],
        out_specs=pl.BlockSpec((TB, INF), lambda tb, h: (tb, 0)),
        out_shape=jax.ShapeDtypeStruct((SEQ, INF), jnp.float32),
        compiler_params=pltpu.CompilerParams(
            dimension_semantics=("arbitrary", "arbitrary")),
    )(inp2d, inp_bf, W_res, wq, wl, wr, wk, embin, embout)
    return out.reshape(1, SEQ, INF)


# speedup vs baseline: 13.3623x; 1.4437x over previous
import jax
import jax.numpy as jnp
from jax.experimental import pallas as pl
from jax.experimental.pallas import tpu as pltpu

NHEAD = 8
QDIM = 512
TOPK = 8
NQ = 256
SEQ = 2048
INF = 1024
TB = 256
NTB = SEQ // TB
EMB_ROWS = (NQ - 1) * TOPK + (NQ - 1) + 1
EMB_PAD = 2304

_SQRT_2_OVER_PI = 0.7978845608028654


def _top8_packed(s, nbits):
    rows, n = s.shape
    mask = (1 << nbits) - 1
    iota = jax.lax.broadcasted_iota(jnp.int32, (rows, n), 1)
    si = jax.lax.bitcast_convert_type(s, jnp.int32)
    ki = jnp.bitwise_or(jnp.bitwise_and(si, jnp.int32(~mask)), mask - iota)
    key = jax.lax.bitcast_convert_type(ki, jnp.float32)
    vals, idxs = [], []
    for _ in range(TOPK):
        m = jnp.max(key, axis=-1, keepdims=True)
        key = jnp.where(key == m, -jnp.inf, key)
        mb = jax.lax.bitcast_convert_type(m, jnp.int32)
        vals.append(m)
        idxs.append(mask - jnp.bitwise_and(mb, jnp.int32(mask)))
    return jnp.concatenate(vals, axis=1), jnp.concatenate(idxs, axis=1)


def _nt(a, b):
    return jax.lax.dot_general(a, b, (((1,), (1,)), ((), ())),
                               preferred_element_type=jnp.float32)


def _fused_body(inp_f32, inp_bf, wres, wq, wl, wr, wk, embin, embout,
                out_ref):
    h = pl.program_id(1)

    x = _nt(inp_bf[...], wq[0])
    xb = x.astype(jnp.bfloat16)
    sl = _nt(xb, wl[...])
    sr = _nt(xb, wr[...])

    lv, li = _top8_packed(sl, 8)
    rv, ri = _top8_packed(sr, 8)

    lrep = jnp.concatenate(
        [jnp.broadcast_to(lv[:, a:a + 1], (TB, TOPK)) for a in range(TOPK)],
        axis=1)
    rtil = jnp.concatenate([rv] * TOPK, axis=1)
    lirep = jnp.concatenate(
        [jnp.broadcast_to(li[:, a:a + 1], (TB, TOPK)) for a in range(TOPK)],
        axis=1)
    ritil = jnp.concatenate([ri] * TOPK, axis=1)
    payload = jnp.bitwise_or(jnp.left_shift(lirep, 8), ritil)
    ci = jnp.bitwise_or(
        jnp.bitwise_and(jax.lax.bitcast_convert_type(lrep + rtil, jnp.int32),
                        jnp.int32(~0xFFFF)), payload)
    ckey = jax.lax.bitcast_convert_type(ci, jnp.float32)

    dots, fibits = [], []
    for _ in range(TOPK):
        m = jnp.max(ckey, axis=-1, keepdims=True)
        ckey = jnp.where(ckey == m, -jnp.inf, ckey)
        dots.append(m)
        fibits.append(jax.lax.bitcast_convert_type(m, jnp.int32))
    dot = jnp.concatenate(dots, axis=1)
    fib = jnp.concatenate(fibits, axis=1)
    fi = (jnp.bitwise_and(jnp.right_shift(fib, 8), 0xFF) * TOPK
          + jnp.bitwise_and(fib, 0xFF))

    e = jnp.exp(dot - jnp.max(dot, axis=-1, keepdims=True))
    scores = e / jnp.sum(e, axis=-1, keepdims=True)

    proj = _nt(inp_bf[...], wk[0])
    ad = _nt(proj.astype(jnp.bfloat16), embin[...])

    iota_e = jax.lax.broadcasted_iota(jnp.int32, (TB, EMB_PAD), 1)
    indots = []
    for k in range(TOPK):
        onek = iota_e == fi[:, k:k + 1]
        indots.append(jnp.sum(jnp.where(onek, ad, 0.0), axis=-1,
                              keepdims=True))
    in_dot = jnp.concatenate(indots, axis=1)

    g = 0.5 * in_dot * (1.0 + jnp.tanh(
        _SQRT_2_OVER_PI * (in_dot + 0.044715 * in_dot * in_dot * in_dot)))
    w = scores * g

    p = jnp.zeros((TB, EMB_PAD), jnp.float32)
    for k in range(TOPK):
        p = p + jnp.where(iota_e == fi[:, k:k + 1], w[:, k:k + 1], 0.0)

    outc = jax.lax.dot_general(p.astype(jnp.bfloat16), embout[...],
                               (((1,), (0,)), ((), ())),
                               preferred_element_type=jnp.float32)

    @pl.when(h == 0)
    def _():
        out_ref[...] = _nt(inp_f32[...], wres[...]) + outc

    @pl.when(h != 0)
    def _():
        out_ref[...] += outc


def kernel(inp, W_res, W_q, W_k, W_left, W_right, emb_in, emb_out):
    inp2d = inp.reshape(SEQ, INF)
    inp_bf = inp2d.astype(jnp.bfloat16)
    wq = W_q.reshape(NHEAD, QDIM, INF).astype(jnp.bfloat16)
    wk = W_k.reshape(NHEAD, INF, INF).astype(jnp.bfloat16)
    wl = W_left.astype(jnp.bfloat16)
    wr = W_right.astype(jnp.bfloat16)
    pad = EMB_PAD - EMB_ROWS
    embin = jnp.pad(emb_in[:EMB_ROWS].astype(jnp.bfloat16), ((0, pad), (0, 0)))
    embout = jnp.pad(emb_out[:EMB_ROWS].astype(jnp.bfloat16),
                     ((0, pad), (0, 0)))

    grid = (NTB, NHEAD)
    out = pl.pallas_call(
        _fused_body,
        grid=grid,
        in_specs=[
            pl.BlockSpec((TB, INF), lambda tb, h: (tb, 0)),
            pl.BlockSpec((TB, INF), lambda tb, h: (tb, 0)),
            pl.BlockSpec((INF, INF), lambda tb, h: (0, 0)),
            pl.BlockSpec((1, QDIM, INF), lambda tb, h: (h, 0, 0)),
            pl.BlockSpec((NQ, QDIM), lambda tb, h: (0, 0)),
            pl.BlockSpec((NQ, QDIM), lambda tb, h: (0, 0)),
            pl.BlockSpec((1, INF, INF), lambda tb, h: (h, 0, 0)),
            pl.BlockSpec((EMB_PAD, INF), lambda tb, h: (0, 0)),
            pl.BlockSpec((EMB_PAD, INF), lambda tb, h: (0, 0)),
        ],
        out_specs=pl.BlockSpec((TB, INF), lambda tb, h: (tb, 0)),
        out_shape=jax.ShapeDtypeStruct((SEQ, INF), jnp.float32),
        compiler_params=pltpu.CompilerParams(
            dimension_semantics=("arbitrary", "arbitrary")),
    )(inp2d, inp_bf, W_res, wq, wl, wr, wk, embin, embout)
    return out.reshape(1, SEQ, INF)


# speedup vs baseline: 14.9287x; 1.1172x over previous
import jax
import jax.numpy as jnp
from jax.experimental import pallas as pl
from jax.experimental.pallas import tpu as pltpu

NHEAD = 8
QDIM = 512
TOPK = 8
NQ = 256
SEQ = 2048
INF = 1024
TB = 256
NTB = SEQ // TB
EMB_ROWS = (NQ - 1) * TOPK + (NQ - 1) + 1
EMB_PAD = 2304

_SQRT_2_OVER_PI = 0.7978845608028654


def _top8_packed(s, nbits):
    rows, n = s.shape
    mask = (1 << nbits) - 1
    iota = jax.lax.broadcasted_iota(jnp.int32, (rows, n), 1)
    si = jax.lax.bitcast_convert_type(s, jnp.int32)
    ki = jnp.bitwise_or(jnp.bitwise_and(si, jnp.int32(~mask)), mask - iota)
    key = jax.lax.bitcast_convert_type(ki, jnp.float32)
    vals, idxs = [], []
    for _ in range(TOPK):
        m = jnp.max(key, axis=-1, keepdims=True)
        key = jnp.where(key == m, -jnp.inf, key)
        mb = jax.lax.bitcast_convert_type(m, jnp.int32)
        vals.append(m)
        idxs.append(mask - jnp.bitwise_and(mb, jnp.int32(mask)))
    return jnp.concatenate(vals, axis=1), jnp.concatenate(idxs, axis=1)


def _nt(a, b):
    return jax.lax.dot_general(a, b, (((1,), (1,)), ((), ())),
                               preferred_element_type=jnp.float32)


def _fused_body(inp_f32, inp_bf, wres, wq, wl, wr, wk, embin, embout,
                out_ref):
    h = pl.program_id(1)

    x = _nt(inp_bf[...], wq[0])
    xb = x.astype(jnp.bfloat16)
    sl = _nt(xb, wl[...])
    sr = _nt(xb, wr[...])

    lv, li = _top8_packed(sl, 8)
    rv, ri = _top8_packed(sr, 8)

    lrep = jnp.concatenate(
        [jnp.broadcast_to(lv[:, a:a + 1], (TB, TOPK)) for a in range(TOPK)],
        axis=1)
    rtil = jnp.concatenate([rv] * TOPK, axis=1)
    lirep = jnp.concatenate(
        [jnp.broadcast_to(li[:, a:a + 1], (TB, TOPK)) for a in range(TOPK)],
        axis=1)
    ritil = jnp.concatenate([ri] * TOPK, axis=1)
    payload = jnp.bitwise_or(jnp.left_shift(lirep, 8), ritil)
    ci = jnp.bitwise_or(
        jnp.bitwise_and(jax.lax.bitcast_convert_type(lrep + rtil, jnp.int32),
                        jnp.int32(~0xFFFF)), payload)
    ckey = jax.lax.bitcast_convert_type(ci, jnp.float32)

    dots, fibits = [], []
    for _ in range(TOPK):
        m = jnp.max(ckey, axis=-1, keepdims=True)
        ckey = jnp.where(ckey == m, -jnp.inf, ckey)
        dots.append(m)
        fibits.append(jax.lax.bitcast_convert_type(m, jnp.int32))
    dot = jnp.concatenate(dots, axis=1)
    fib = jnp.concatenate(fibits, axis=1)
    fi = (jnp.bitwise_and(jnp.right_shift(fib, 8), 0xFF) * TOPK
          + jnp.bitwise_and(fib, 0xFF))

    e = jnp.exp(dot - jnp.max(dot, axis=-1, keepdims=True))
    scores = e / jnp.sum(e, axis=-1, keepdims=True)

    proj = _nt(inp_bf[...], wk[0])
    ad = _nt(proj.astype(jnp.bfloat16), embin[...])

    lane = jnp.bitwise_and(fi, 127)
    bsel = jnp.right_shift(fi, 7)
    in_dot = jnp.zeros((TB, TOPK), jnp.float32)
    for j in range(EMB_PAD // 128):
        g = jnp.take_along_axis(ad[:, j * 128:(j + 1) * 128], lane, axis=1,
                                mode='promise_in_bounds')
        in_dot = in_dot + jnp.where(bsel == j, g, 0.0)

    g = 0.5 * in_dot * (1.0 + jnp.tanh(
        _SQRT_2_OVER_PI * (in_dot + 0.044715 * in_dot * in_dot * in_dot)))
    w = scores * g

    iota_e = jax.lax.broadcasted_iota(jnp.int16, (TB, EMB_PAD), 1)
    fi16 = fi.astype(jnp.int16)
    w16 = w.astype(jnp.bfloat16)
    p = jnp.zeros((TB, EMB_PAD), jnp.bfloat16)
    for k in range(TOPK):
        p = p + jnp.where(iota_e == fi16[:, k:k + 1], w16[:, k:k + 1],
                          jnp.bfloat16(0.0))

    outc = jax.lax.dot_general(p, embout[...],
                               (((1,), (0,)), ((), ())),
                               preferred_element_type=jnp.float32)

    @pl.when(h == 0)
    def _():
        out_ref[...] = _nt(inp_f32[...], wres[...]) + outc

    @pl.when(h != 0)
    def _():
        out_ref[...] += outc


def kernel(inp, W_res, W_q, W_k, W_left, W_right, emb_in, emb_out):
    inp2d = inp.reshape(SEQ, INF)
    inp_bf = inp2d.astype(jnp.bfloat16)
    wq = W_q.reshape(NHEAD, QDIM, INF).astype(jnp.bfloat16)
    wk = W_k.reshape(NHEAD, INF, INF).astype(jnp.bfloat16)
    wl = W_left.astype(jnp.bfloat16)
    wr = W_right.astype(jnp.bfloat16)
    pad = EMB_PAD - EMB_ROWS
    embin = jnp.pad(emb_in[:EMB_ROWS].astype(jnp.bfloat16), ((0, pad), (0, 0)))
    embout = jnp.pad(emb_out[:EMB_ROWS].astype(jnp.bfloat16),
                     ((0, pad), (0, 0)))

    grid = (NTB, NHEAD)
    out = pl.pallas_call(
        _fused_body,
        grid=grid,
        in_specs=[
            pl.BlockSpec((TB, INF), lambda tb, h: (tb, 0)),
            pl.BlockSpec((TB, INF), lambda tb, h: (tb, 0)),
            pl.BlockSpec((INF, INF), lambda tb, h: (0, 0)),
            pl.BlockSpec((1, QDIM, INF), lambda tb, h: (h, 0, 0)),
            pl.BlockSpec((NQ, QDIM), lambda tb, h: (0, 0)),
            pl.BlockSpec((NQ, QDIM), lambda tb, h: (0, 0)),
            pl.BlockSpec((1, INF, INF), lambda tb, h: (h, 0, 0)),
            pl.BlockSpec((EMB_PAD, INF), lambda tb, h: (0, 0)),
            pl.BlockSpec((EMB_PAD, INF), lambda tb, h: (0, 0)),
        ],
        out_specs=pl.BlockSpec((TB, INF), lambda tb, h: (tb, 0)),
        out_shape=jax.ShapeDtypeStruct((SEQ, INF), jnp.float32),
        compiler_params=pltpu.CompilerParams(
            dimension_semantics=("arbitrary", "arbitrary")),
    )(inp2d, inp_bf, W_res, wq, wl, wr, wk, embin, embout)
    return out.reshape(1, SEQ, INF)
